# baseline (speedup 1.0000x reference)
import jax
import jax.numpy as jnp
from jax import lax
from jax.experimental import pallas as pl
from jax.experimental.pallas import tpu as pltpu
from jax.experimental.pallas import tpu_sc as plsc

M_ITEM = 100000
DIM = 32
N_CLIENTS = 26
N_ITEMS = 200
LR = 0.01

NW = 32
ROWS_MAIN = 3120
CHUNK = 240
N_CHUNKS = ROWS_MAIN // CHUNK
NBUF = 3
TAIL_BASE = ROWS_MAIN * NW
TAIL_ROWS = 8
N_TAIL_W = (M_ITEM - TAIL_BASE) // TAIL_ROWS
N_ITEMS_PAD = 208
N_GROUPS = N_ITEMS_PAD // 16
C_PAD = 32


def _patch_chunk(items_v, grad_v, buf, lane, lo, hi):
    def group_body(g, carry):
        iv = items_v[pl.ds(g * 16, 16)]
        in_c = (iv >= lo) & (iv < hi)
        cnt = jnp.max(plsc.all_reduce_population_count(in_c))

        @pl.when(cnt > 0)
        def _():
            rows = jnp.where(in_c, iv - lo, 0)
            kvec = lane + g * 16
            for c in range(DIM):
                colv = jnp.full((16,), c, jnp.int32)
                gval = plsc.load_gather(grad_v, [kvec, colv], mask=in_c)
                cur = plsc.load_gather(buf, [rows, colv], mask=in_c)
                plsc.store_scatter(buf, [rows, colv], cur - LR * gval,
                                   mask=in_c)
        return carry

    hit = jnp.int32(0)
    for g in range(N_GROUPS):
        iv = items_v[pl.ds(g * 16, 16)]
        in_c = (iv >= lo) & (iv < hi)
        hit = hit | jnp.max(plsc.all_reduce_population_count(in_c))

    @pl.when(hit > 0)
    def _():
        lax.fori_loop(0, N_GROUPS, group_body, jnp.int32(0))


def _body(items_hbm, grad_hbm, losses_hbm, weight_hbm, out_hbm,
          losses_v, items_v, grad_v, buf0, buf1, buf2, tbuf,
          in_sem0, in_sem1, in_sem2, out_sem0, out_sem1, out_sem2,
          tail_sem):
    wid = lax.axis_index("s") * 2 + lax.axis_index("c")
    base_a = wid * ROWS_MAIN
    base_b = TAIL_BASE + wid * TAIL_ROWS
    has_tail = wid < N_TAIL_W
    bufs = (buf0, buf1, buf2)
    in_sems = (in_sem0, in_sem1, in_sem2)
    out_sems = (out_sem0, out_sem1, out_sem2)

    def in_cp(ci):
        return pltpu.make_async_copy(
            weight_hbm.at[pl.ds(base_a + ci * CHUNK, CHUNK)],
            bufs[ci % NBUF], in_sems[ci % NBUF])

    def out_cp(ci):
        return pltpu.make_async_copy(
            bufs[ci % NBUF], out_hbm.at[pl.ds(base_a + ci * CHUNK, CHUNK)],
            out_sems[ci % NBUF])

    in_cp(0).start()
    in_cp(1).start()
    in_cp(2).start()

    tail_in = pltpu.make_async_copy(
        weight_hbm.at[pl.ds(base_b, TAIL_ROWS)], tbuf, tail_sem)

    @pl.when(has_tail)
    def _tail_start():
        tail_in.start()

    pltpu.sync_copy(losses_hbm, losses_v)
    lane = lax.iota(jnp.int32, 16)
    v0 = losses_v[pl.ds(0, 16)]
    v1 = losses_v[pl.ds(16, 16)]
    valid1 = lane < (N_CLIENTS - 16)
    half = (jnp.sum(jnp.where(valid1, v1, 0.0)) + jnp.sum(v0)) * 0.5

    c_star = jnp.int32(0)
    for i in range(N_CLIENTS):
        src = v0 if i < 16 else v1
        li = jnp.sum(jnp.where(lane == (i % 16), src, 0.0))
        p0 = (v0 < li) | ((v0 == li) & (lane < i))
        p1 = (v1 < li) | ((v1 == li) & ((lane + 16) < i))
        s = jnp.sum(jnp.where(p0, v0, 0.0)) + jnp.sum(jnp.where(p1, v1, 0.0))
        sel = (s < half) & (s + li >= half)
        c_star = jnp.where(sel, jnp.int32(i), c_star)

    pltpu.sync_copy(items_hbm.at[pl.ds(c_star * N_ITEMS_PAD, N_ITEMS_PAD)],
                    items_v)
    pltpu.sync_copy(grad_hbm.at[c_star], grad_v.at[pl.ds(0, N_ITEMS)])

    for ci in range(N_CHUNKS):
        if ci >= 1 and ci + 2 < N_CHUNKS:
            out_cp(ci - 1).wait()
            in_cp(ci + 2).start()
        in_cp(ci).wait()
        _patch_chunk(items_v, grad_v, bufs[ci % NBUF], lane,
                     base_a + ci * CHUNK, base_a + (ci + 1) * CHUNK)
        out_cp(ci).start()

    @pl.when(has_tail)
    def _tail_done():
        tail_in.wait()
        _patch_chunk(items_v, grad_v, tbuf, lane, base_b, base_b + TAIL_ROWS)
        pltpu.sync_copy(tbuf, out_hbm.at[pl.ds(base_b, TAIL_ROWS)])

    out_cp(N_CHUNKS - 3).wait()
    out_cp(N_CHUNKS - 2).wait()
    out_cp(N_CHUNKS - 1).wait()


@jax.jit
def _run(items_p, grads, losses_p, weight):
    mesh = plsc.VectorSubcoreMesh(core_axis_name="c", subcore_axis_name="s",
                                  num_cores=2, num_subcores=16)
    return pl.kernel(
        _body,
        out_type=jax.ShapeDtypeStruct((M_ITEM, DIM), jnp.float32),
        mesh=mesh,
        compiler_params=pltpu.CompilerParams(needs_layout_passes=False),
        scratch_types=[
            pltpu.VMEM((C_PAD,), jnp.float32),
            pltpu.VMEM((N_ITEMS_PAD,), jnp.int32),
            pltpu.VMEM((N_ITEMS_PAD, DIM), jnp.float32),
            pltpu.VMEM((CHUNK, DIM), jnp.float32),
            pltpu.VMEM((CHUNK, DIM), jnp.float32),
            pltpu.VMEM((CHUNK, DIM), jnp.float32),
            pltpu.VMEM((TAIL_ROWS, DIM), jnp.float32),
            pltpu.SemaphoreType.DMA,
            pltpu.SemaphoreType.DMA,
            pltpu.SemaphoreType.DMA,
            pltpu.SemaphoreType.DMA,
            pltpu.SemaphoreType.DMA,
            pltpu.SemaphoreType.DMA,
            pltpu.SemaphoreType.DMA,
        ],
    )(items_p, grads, losses_p, weight)


def kernel(items, items_emb_grad, client_losses, items_emb_weight):
    items_p = jnp.pad(items, ((0, 0), (0, N_ITEMS_PAD - N_ITEMS)),
                      constant_values=-1).reshape(-1)
    losses_p = jnp.pad(client_losses, (0, C_PAD - N_CLIENTS),
                       constant_values=jnp.inf)
    return _run(items_p, items_emb_grad, losses_p, items_emb_weight)


# speedup vs baseline: 3.6964x; 3.6964x over previous
import jax
import jax.numpy as jnp
from jax import lax
from jax.experimental import pallas as pl
from jax.experimental.pallas import tpu as pltpu
from jax.experimental.pallas import tpu_sc as plsc

M_ITEM = 100000
DIM = 32
N_CLIENTS = 26
N_ITEMS = 200
LR = 0.01

NW = 32
COLS_MAIN = 3072
EXTRA_BASE = COLS_MAIN * NW
N_EXTRA_W = 13
FINAL_BASE = EXTRA_BASE + N_EXTRA_W * 128
FINAL_COLS = M_ITEM - FINAL_BASE
FINAL_W = 13
N_ITEMS_PAD = 208
N_GROUPS = N_ITEMS_PAD // 16
C_PAD = 32


def _patch(items_v, grad_v, buf, lane, lo, width):
    def group_body(g, carry):
        iv = items_v[pl.ds(g * 16, 16)]
        in_c = (iv >= lo) & (iv < lo + width)
        cnt = jnp.max(plsc.all_reduce_population_count(in_c))

        @pl.when(cnt > 0)
        def _():
            cols = jnp.where(in_c, iv - lo, 0)
            kvec = jnp.minimum(lane + g * 16, N_ITEMS - 1)
            for c in range(DIM):
                rowv = jnp.full((16,), c, jnp.int32)
                gval = plsc.load_gather(grad_v, [rowv, kvec], mask=in_c)
                cur = plsc.load_gather(buf, [rowv, cols], mask=in_c)
                plsc.store_scatter(buf, [rowv, cols], cur - LR * gval,
                                   mask=in_c)
        return carry

    hit = jnp.int32(0)
    for g in range(N_GROUPS):
        iv = items_v[pl.ds(g * 16, 16)]
        in_c = (iv >= lo) & (iv < lo + width)
        hit = hit | jnp.max(plsc.all_reduce_population_count(in_c))

    @pl.when(hit > 0)
    def _():
        lax.fori_loop(0, N_GROUPS, group_body, jnp.int32(0))


def _body(items_hbm, grad_hbm, losses_hbm, weight_hbm, out_hbm,
          losses_v, items_v, grad_v, slab_v, ebuf, fbuf,
          slab_sem, extra_sem):
    wid = lax.axis_index("s") * 2 + lax.axis_index("c")
    base = wid * COLS_MAIN
    ebase = EXTRA_BASE + wid * 128
    has_extra = wid < N_EXTRA_W
    has_final = wid == FINAL_W

    slab_cp = pltpu.make_async_copy(
        weight_hbm.at[:, pl.ds(base, COLS_MAIN)], slab_v, slab_sem)
    slab_cp.start()

    extra_cp = pltpu.make_async_copy(
        weight_hbm.at[:, pl.ds(ebase, 128)], ebuf, extra_sem)
    final_cp = pltpu.make_async_copy(
        weight_hbm.at[:, pl.ds(FINAL_BASE, FINAL_COLS)], fbuf, extra_sem)

    @pl.when(has_extra)
    def _extra_start():
        extra_cp.start()

    @pl.when(has_final)
    def _final_start():
        final_cp.start()

    pltpu.sync_copy(losses_hbm, losses_v)
    lane = lax.iota(jnp.int32, 16)
    v0 = losses_v[pl.ds(0, 16)]
    v1 = losses_v[pl.ds(16, 16)]
    valid1 = lane < (N_CLIENTS - 16)
    half = (jnp.sum(jnp.where(valid1, v1, 0.0)) + jnp.sum(v0)) * 0.5

    c_star = jnp.int32(0)
    for i in range(N_CLIENTS):
        src = v0 if i < 16 else v1
        li = jnp.sum(jnp.where(lane == (i % 16), src, 0.0))
        p0 = (v0 < li) | ((v0 == li) & (lane < i))
        p1 = (v1 < li) | ((v1 == li) & ((lane + 16) < i))
        s = jnp.sum(jnp.where(p0, v0, 0.0)) + jnp.sum(jnp.where(p1, v1, 0.0))
        sel = (s < half) & (s + li >= half)
        c_star = jnp.where(sel, jnp.int32(i), c_star)

    pltpu.sync_copy(items_hbm.at[pl.ds(c_star * N_ITEMS_PAD, N_ITEMS_PAD)],
                    items_v)
    pltpu.sync_copy(grad_hbm.at[c_star], grad_v)

    slab_cp.wait()
    _patch(items_v, grad_v, slab_v, lane, base, COLS_MAIN)
    out_cp = pltpu.make_async_copy(
        slab_v, out_hbm.at[:, pl.ds(base, COLS_MAIN)], slab_sem)
    out_cp.start()

    @pl.when(has_extra)
    def _extra_done():
        extra_cp.wait()
        _patch(items_v, grad_v, ebuf, lane, ebase, 128)
        pltpu.sync_copy(ebuf, out_hbm.at[:, pl.ds(ebase, 128)])

    @pl.when(has_final)
    def _final_done():
        final_cp.wait()
        _patch(items_v, grad_v, fbuf, lane, FINAL_BASE, FINAL_COLS)
        pltpu.sync_copy(fbuf, out_hbm.at[:, pl.ds(FINAL_BASE, FINAL_COLS)])

    out_cp.wait()


@jax.jit
def _run(items_p, grads_t, losses_p, weight_t):
    mesh = plsc.VectorSubcoreMesh(core_axis_name="c", subcore_axis_name="s",
                                  num_cores=2, num_subcores=16)
    return pl.kernel(
        _body,
        out_type=jax.ShapeDtypeStruct((DIM, M_ITEM), jnp.float32),
        mesh=mesh,
        compiler_params=pltpu.CompilerParams(needs_layout_passes=False),
        scratch_types=[
            pltpu.VMEM((C_PAD,), jnp.float32),
            pltpu.VMEM((N_ITEMS_PAD,), jnp.int32),
            pltpu.VMEM((DIM, N_ITEMS), jnp.float32),
            pltpu.VMEM((DIM, COLS_MAIN), jnp.float32),
            pltpu.VMEM((DIM, 128), jnp.float32),
            pltpu.VMEM((DIM, FINAL_COLS), jnp.float32),
            pltpu.SemaphoreType.DMA,
            pltpu.SemaphoreType.DMA,
        ],
    )(items_p, grads_t, losses_p, weight_t)


def kernel(items, items_emb_grad, client_losses, items_emb_weight):
    items_p = jnp.pad(items, ((0, 0), (0, N_ITEMS_PAD - N_ITEMS)),
                      constant_values=-1).reshape(-1)
    losses_p = jnp.pad(client_losses, (0, C_PAD - N_CLIENTS),
                       constant_values=jnp.inf)
    grads_t = jnp.transpose(items_emb_grad, (0, 2, 1))
    weight_t = items_emb_weight.T
    out_t = _run(items_p, grads_t, losses_p, weight_t)
    return out_t.T
